# baseline (speedup 1.0000x reference)
import jax
import jax.numpy as jnp
from jax.experimental import pallas as pl
from jax.experimental.pallas import tpu as pltpu


def _to_blocks(t):
    *lead, H, W, C = t.shape
    t = jnp.pad(t, [(0, 0)] * len(lead) + [(1, 1), (1, 1), (0, 0)])
    t = t.reshape(*lead, (H + 2) // 2, 2, (W + 2) // 2, 2, C)
    t = jnp.moveaxis(t, -4, -3)
    return t.reshape(*lead, (H + 2) // 2, (W + 2) // 2, 4 * C)


def _blocks_to_image(t):
    *lead, Hb, Wb, C4 = t.shape
    C = C4 // 4
    t = t.reshape(*lead, Hb, Wb, 2, 2, C)
    t = jnp.moveaxis(t, -3, -4)
    return t.reshape(*lead, 2 * Hb, 2 * Wb, C)


def _make_ingest_body(C, rs2, W, H):
    Wb1 = W // 2 + 1

    def body(m_ref, p_ref, o_ref):
        r = pl.program_id(1)
        bf = jnp.bfloat16
        E = m_ref[0, :, :, 0, :].astype(bf)
        Od = m_ref[0, :, :, 1, :].astype(bf)
        Op = p_ref[0, :, :, 1, :].astype(bf)
        Op = Op * (r > 0).astype(bf)
        OA = jnp.concatenate([Op, Od[:, :-1, :]], axis=1)
        Et = jnp.transpose(E, (1, 2, 0))
        Ot = jnp.transpose(OA, (1, 2, 0))
        pid = r * rs2 + jax.lax.broadcasted_iota(jnp.int32, (rs2, 1, 1), 0)
        Et = jnp.where(pid >= H // 2, jnp.zeros_like(Et), Et)
        z = jnp.zeros((rs2, 1, C), bf)
        Ev = jnp.concatenate([z, Et, z], axis=1).reshape(rs2, Wb1, 2, C)
        Ov = jnp.concatenate([z, Ot, z], axis=1).reshape(rs2, Wb1, 2, C)
        o_ref[0] = jnp.stack([Ov, Ev], axis=2)

    return body


def _ingest_bgs(img, *, rs2=16):
    B, C, H, W = img.shape
    S = (H // 2 + 1 + rs2 - 1) // rs2
    ar = img.reshape(B, C, H // 2, 2, W)
    out = pl.pallas_call(
        _make_ingest_body(C, rs2, W, H),
        out_shape=jax.ShapeDtypeStruct((B, S * rs2, W // 2 + 1, 2, 2, C),
                                       jnp.bfloat16),
        grid=(B, S),
        in_specs=[
            pl.BlockSpec((1, C, rs2, 2, W),
                         lambda i, r: (i, 0, jnp.minimum(r, H // 2 // rs2 - 1), 0, 0)),
            pl.BlockSpec((1, C, 1, 2, W),
                         lambda i, r: (i, 0, jnp.maximum(r * rs2 - 1, 0), 0, 0)),
        ],
        out_specs=pl.BlockSpec((1, rs2, W // 2 + 1, 2, 2, C),
                               lambda i, r: (i, r, 0, 0, 0, 0)),
        compiler_params=pltpu.CompilerParams(
            dimension_semantics=("parallel", "arbitrary")),
    )(ar, ar)
    return out.reshape(B, S * rs2, W // 2 + 1, 4 * C)


def _make_conv_body(ts, Wout, C4, Cout, pool):
    N4 = 4 * Cout
    M = ts * Wout
    Mm = M - Wout

    def body(main_ref, extra_ref, w_ref, b_ref, o_ref):
        win = jnp.concatenate([main_ref[0], extra_ref[0]], axis=0)
        f0 = win[:, 0:Wout, :].reshape((ts + 1) * Wout, C4)
        f1 = win[:, 1:1 + Wout, :].reshape((ts + 1) * Wout, C4)
        acc = jnp.broadcast_to(b_ref[0], (M, N4))
        acc = acc + jnp.dot(f0[0:M], w_ref[0, 0], preferred_element_type=jnp.float32)
        acc = acc + jnp.dot(f1[0:M], w_ref[0, 1], preferred_element_type=jnp.float32)
        acc = acc + jnp.dot(f0[Wout:Wout + M], w_ref[0, 2],
                            preferred_element_type=jnp.float32)
        acc = acc + jnp.dot(f1[Wout:Wout + M], w_ref[0, 3],
                            preferred_element_type=jnp.float32)
        acc = jnp.maximum(acc, 0.0)
        if pool:
            y = jnp.maximum(
                jnp.maximum(acc[:, 0:Cout], acc[:, Cout:2 * Cout]),
                jnp.maximum(acc[:, 2 * Cout:3 * Cout], acc[:, 3 * Cout:4 * Cout]))
            o_ref[0] = y.reshape(ts, Wout, Cout).astype(o_ref.dtype)
        else:
            o_ref[0] = acc.reshape(ts, Wout, N4).astype(o_ref.dtype)

    return body


def _conv_layer(xb, w_stack, b_stack, *, batch, Cout, pool, ts, hout=None):
    TB, HB, WB, C4 = xb.shape
    Hout, Wout = (hout if hout is not None else HB - 1), WB - 1
    N4 = 4 * Cout
    Nout = Cout if pool else N4
    n_strips = Hout // ts
    body = _make_conv_body(ts, Wout, C4, Cout, pool)
    return pl.pallas_call(
        body,
        out_shape=jax.ShapeDtypeStruct((TB, Hout, Wout, Nout), jnp.bfloat16),
        grid=(TB, n_strips),
        in_specs=[
            pl.BlockSpec((1, ts, WB, C4), lambda i, r: (i, r, 0, 0)),
            pl.BlockSpec((1, 1, WB, C4), lambda i, r: (i, (r + 1) * ts, 0, 0)),
            pl.BlockSpec((1, 4, C4, N4), lambda i, r: (i // batch, 0, 0, 0)),
            pl.BlockSpec((1, 1, N4), lambda i, r: (i // batch, 0, 0)),
        ],
        out_specs=pl.BlockSpec((1, ts, Wout, Nout), lambda i, r: (i, r, 0, 0)),
        compiler_params=pltpu.CompilerParams(
            dimension_semantics=("parallel", "arbitrary")),
    )(xb, xb, w_stack, b_stack)


def _fc1_body(f_ref, w_ref, b_ref, o_ref, acc_ref):
    k = pl.program_id(1)

    @pl.when(k == 0)
    def _init():
        acc_ref[...] = jnp.broadcast_to(b_ref[0], acc_ref.shape)

    acc_ref[...] += jnp.dot(f_ref[0], w_ref[0].astype(jnp.bfloat16),
                            preferred_element_type=jnp.float32)

    @pl.when(k == pl.num_programs(1) - 1)
    def _finish():
        o_ref[0] = jnp.maximum(acc_ref[...], 0.0).astype(o_ref.dtype)


def _fc1_apply(feats, w, b, *, tk=4096):
    S, B, K = feats.shape
    N = w.shape[-1]
    return pl.pallas_call(
        _fc1_body,
        out_shape=jax.ShapeDtypeStruct((S, B, N), jnp.bfloat16),
        grid=(S, K // tk),
        in_specs=[
            pl.BlockSpec((1, B, tk), lambda s, k: (s, 0, k)),
            pl.BlockSpec((1, tk, N), lambda s, k: (s, k, 0)),
            pl.BlockSpec((1, 1, N), lambda s, k: (s, 0, 0)),
        ],
        out_specs=pl.BlockSpec((1, B, N), lambda s, k: (s, 0, 0)),
        scratch_shapes=[pltpu.VMEM((B, N), jnp.float32)],
        compiler_params=pltpu.CompilerParams(
            dimension_semantics=("parallel", "arbitrary")),
    )(feats, w, b)


def _head_body(f_ref, w1_ref, b1_ref, w2_ref, b2_ref, o_ref):
    w1 = w1_ref[...]
    nh = w1.shape[0] // 2
    hv = (jnp.dot(f_ref[0], w1[0:nh].astype(jnp.bfloat16),
                  preferred_element_type=jnp.float32)
          + jnp.dot(f_ref[1], w1[nh:2 * nh].astype(jnp.bfloat16),
                    preferred_element_type=jnp.float32)
          + b1_ref[...])
    hv = jnp.maximum(hv, 0.0)
    o_ref[...] = jnp.sum(hv * w2_ref[...], axis=1, keepdims=True) + b2_ref[...]


def _head_apply(feats, w1, b1, w2, b2):
    B = feats.shape[1]
    return pl.pallas_call(
        _head_body,
        out_shape=jax.ShapeDtypeStruct((B, 1), jnp.float32),
    )(feats, w1, b1.reshape(1, -1), w2.reshape(1, -1), b2.reshape(1, 1))


def kernel(x, a, conv21_w, conv21_b,
           lenet_w_0, lenet_w_1, lenet_w_2, lenet_w_3,
           lenet_b_0, lenet_b_1, lenet_b_2, lenet_b_3,
           fc1_w, fc1_b, head_w1, head_b1, head_w2, head_b2):
    B = x.shape[0]
    bf = jnp.bfloat16

    a_bgs = _ingest_bgs(a)
    a21 = _conv_layer(a_bgs, conv21_w.astype(bf), conv21_b,
                      batch=B, Cout=3, pool=False, ts=16,
                      hout=128)
    a21 = _blocks_to_image(a21)

    xh = jnp.transpose(x, (0, 2, 3, 1)).astype(bf)
    h = jnp.concatenate([_to_blocks(xh), _to_blocks(a21)], axis=0)
    h = _conv_layer(h, lenet_w_0.astype(bf), lenet_b_0,
                    batch=B, Cout=16, pool=True, ts=16)
    lenet_w = [lenet_w_1, lenet_w_2, lenet_w_3]
    lenet_b = [lenet_b_1, lenet_b_2, lenet_b_3]
    cfg = [(32, 32), (64, 32), (128, 16)]
    for i, (cout, ts) in enumerate(cfg):
        h = _conv_layer(_to_blocks(h), lenet_w[i].astype(bf), lenet_b[i],
                        batch=B, Cout=cout, pool=True, ts=ts)

    feats = h.reshape(2, B, 16 * 16 * 128)
    hfc = _fc1_apply(feats, fc1_w, fc1_b)
    return _head_apply(hfc, head_w1, head_b1, head_w2, head_b2)


# speedup vs baseline: 1.2145x; 1.2145x over previous
import jax
import jax.numpy as jnp
from jax.experimental import pallas as pl
from jax.experimental.pallas import tpu as pltpu


def _to_blocks(t):
    *lead, H, W, C = t.shape
    t = jnp.pad(t, [(0, 0)] * len(lead) + [(1, 1), (1, 1), (0, 0)])
    t = t.reshape(*lead, (H + 2) // 2, 2, (W + 2) // 2, 2, C)
    t = jnp.moveaxis(t, -4, -3)
    return t.reshape(*lead, (H + 2) // 2, (W + 2) // 2, 4 * C)


def _blocks_to_image(t):
    *lead, Hb, Wb, C4 = t.shape
    C = C4 // 4
    t = t.reshape(*lead, Hb, Wb, 2, 2, C)
    t = jnp.moveaxis(t, -3, -4)
    return t.reshape(*lead, 2 * Hb, 2 * Wb, C)


def _make_conv_body(ts, Wout, C4, Cout, pool):
    N4 = 4 * Cout
    M = ts * Wout
    Mm = M - Wout

    def body(main_ref, extra_ref, w_ref, b_ref, o_ref):
        win = jnp.concatenate([main_ref[0], extra_ref[0, 0:1]], axis=0)
        f0 = win[:, 0:Wout, :].reshape((ts + 1) * Wout, C4)
        f1 = win[:, 1:1 + Wout, :].reshape((ts + 1) * Wout, C4)
        acc = jnp.broadcast_to(b_ref[0], (M, N4))
        acc = acc + jnp.dot(f0[0:M], w_ref[0, 0], preferred_element_type=jnp.float32)
        acc = acc + jnp.dot(f1[0:M], w_ref[0, 1], preferred_element_type=jnp.float32)
        acc = acc + jnp.dot(f0[Wout:Wout + M], w_ref[0, 2],
                            preferred_element_type=jnp.float32)
        acc = acc + jnp.dot(f1[Wout:Wout + M], w_ref[0, 3],
                            preferred_element_type=jnp.float32)
        acc = jnp.maximum(acc, 0.0)
        if pool:
            y = jnp.maximum(
                jnp.maximum(acc[:, 0:Cout], acc[:, Cout:2 * Cout]),
                jnp.maximum(acc[:, 2 * Cout:3 * Cout], acc[:, 3 * Cout:4 * Cout]))
            o_ref[0] = y.reshape(ts, Wout, Cout).astype(o_ref.dtype)
        else:
            o_ref[0] = acc.reshape(ts, Wout, N4).astype(o_ref.dtype)

    return body


def _conv_layer(xb, w_stack, b_stack, *, batch, Cout, pool, ts, hout=None):
    TB, HB, WB, C4 = xb.shape
    Hout, Wout = (hout if hout is not None else HB - 1), WB - 1
    N4 = 4 * Cout
    Nout = Cout if pool else N4
    assert ts % 8 == 0 and Hout % ts == 0
    n_strips = Hout // ts
    body = _make_conv_body(ts, Wout, C4, Cout, pool)
    return pl.pallas_call(
        body,
        out_shape=jax.ShapeDtypeStruct((TB, Hout, Wout, Nout), jnp.bfloat16),
        grid=(TB, n_strips),
        in_specs=[
            pl.BlockSpec((1, ts, WB, C4), lambda i, r: (i, r, 0, 0)),
            pl.BlockSpec((1, 8, WB, C4), lambda i, r: (i, (r + 1) * ts // 8, 0, 0)),
            pl.BlockSpec((1, 4, C4, N4), lambda i, r: (i // batch, 0, 0, 0)),
            pl.BlockSpec((1, 1, N4), lambda i, r: (i // batch, 0, 0)),
        ],
        out_specs=pl.BlockSpec((1, ts, Wout, Nout), lambda i, r: (i, r, 0, 0)),
        compiler_params=pltpu.CompilerParams(
            dimension_semantics=("parallel", "arbitrary")),
    )(xb, xb, w_stack, b_stack)


def _fc1_body(f_ref, w_ref, b_ref, o_ref, acc_ref):
    k = pl.program_id(1)

    @pl.when(k == 0)
    def _init():
        acc_ref[...] = jnp.broadcast_to(b_ref[0], acc_ref.shape)

    acc_ref[...] += jnp.dot(f_ref[0], w_ref[0].astype(jnp.bfloat16),
                            preferred_element_type=jnp.float32)

    @pl.when(k == pl.num_programs(1) - 1)
    def _finish():
        o_ref[0] = jnp.maximum(acc_ref[...], 0.0).astype(o_ref.dtype)


def _fc1_apply(feats, w, b, *, tk=4096):
    S, B, K = feats.shape
    N = w.shape[-1]
    return pl.pallas_call(
        _fc1_body,
        out_shape=jax.ShapeDtypeStruct((S, B, N), jnp.bfloat16),
        grid=(S, K // tk),
        in_specs=[
            pl.BlockSpec((1, B, tk), lambda s, k: (s, 0, k)),
            pl.BlockSpec((1, tk, N), lambda s, k: (s, k, 0)),
            pl.BlockSpec((1, 1, N), lambda s, k: (s, 0, 0)),
        ],
        out_specs=pl.BlockSpec((1, B, N), lambda s, k: (s, 0, 0)),
        scratch_shapes=[pltpu.VMEM((B, N), jnp.float32)],
        compiler_params=pltpu.CompilerParams(
            dimension_semantics=("parallel", "arbitrary")),
    )(feats, w, b)


def _head_body(f_ref, w1_ref, b1_ref, w2_ref, b2_ref, o_ref):
    w1 = w1_ref[...]
    nh = w1.shape[0] // 2
    hv = (jnp.dot(f_ref[0], w1[0:nh].astype(jnp.bfloat16),
                  preferred_element_type=jnp.float32)
          + jnp.dot(f_ref[1], w1[nh:2 * nh].astype(jnp.bfloat16),
                    preferred_element_type=jnp.float32)
          + b1_ref[...])
    hv = jnp.maximum(hv, 0.0)
    o_ref[...] = jnp.sum(hv * w2_ref[...], axis=1, keepdims=True) + b2_ref[...]


def _head_apply(feats, w1, b1, w2, b2):
    B = feats.shape[1]
    return pl.pallas_call(
        _head_body,
        out_shape=jax.ShapeDtypeStruct((B, 1), jnp.float32),
    )(feats, w1, b1.reshape(1, -1), w2.reshape(1, -1), b2.reshape(1, 1))


def kernel(x, a, conv21_w, conv21_b,
           lenet_w_0, lenet_w_1, lenet_w_2, lenet_w_3,
           lenet_b_0, lenet_b_1, lenet_b_2, lenet_b_3,
           fc1_w, fc1_b, head_w1, head_b1, head_w2, head_b2):
    B = x.shape[0]
    bf = jnp.bfloat16

    ah = jnp.transpose(a, (0, 2, 3, 1)).astype(bf)
    a21 = _conv_layer(_to_blocks(ah), conv21_w.astype(bf), conv21_b,
                      batch=B, Cout=3, pool=False, ts=32)
    a21 = _blocks_to_image(a21)

    xh = jnp.transpose(x, (0, 2, 3, 1)).astype(bf)
    h = jnp.concatenate([_to_blocks(xh), _to_blocks(a21)], axis=0)
    lenet_w = [lenet_w_0, lenet_w_1, lenet_w_2, lenet_w_3]
    lenet_b = [lenet_b_0, lenet_b_1, lenet_b_2, lenet_b_3]
    cfg = [(16, 32), (32, 32), (64, 32), (128, 16)]
    for i, (cout, ts) in enumerate(cfg):
        h = _conv_layer(h, lenet_w[i].astype(bf), lenet_b[i],
                        batch=B, Cout=cout, pool=True, ts=ts)
        if i + 1 < len(cfg):
            h = _to_blocks(h)

    feats = h.reshape(2, B, 16 * 16 * 128)
    hfc = _fc1_apply(feats, fc1_w, fc1_b)
    return _head_apply(hfc, head_w1, head_b1, head_w2, head_b2)


# speedup vs baseline: 1.4449x; 1.1897x over previous
import jax
import jax.numpy as jnp
from jax.experimental import pallas as pl
from jax.experimental.pallas import tpu as pltpu


def _to_blocks(t):
    *lead, H, W, C = t.shape
    t = jnp.pad(t, [(0, 0)] * len(lead) + [(1, 1), (1, 1), (0, 0)])
    t = t.reshape(*lead, (H + 2) // 2, 2, (W + 2) // 2, 2, C)
    t = jnp.moveaxis(t, -4, -3)
    return t.reshape(*lead, (H + 2) // 2, (W + 2) // 2, 4 * C)


def _blocks_to_image(t):
    *lead, Hb, Wb, C4 = t.shape
    C = C4 // 4
    t = t.reshape(*lead, Hb, Wb, 2, 2, C)
    t = jnp.moveaxis(t, -3, -4)
    return t.reshape(*lead, 2 * Hb, 2 * Wb, C)


def _make_conv_body(ts, Wout, C4, Cout, pool):
    N4 = 4 * Cout
    M = ts * Wout
    Mm = M - Wout

    def body(main_ref, extra_ref, w_ref, b_ref, o_ref):
        win = jnp.concatenate([main_ref[0], extra_ref[0, 0:1]], axis=0)
        f0 = win[:, 0:Wout, :].reshape((ts + 1) * Wout, C4)
        f1 = win[:, 1:1 + Wout, :].reshape((ts + 1) * Wout, C4)
        acc = jnp.broadcast_to(b_ref[0], (M, N4))
        acc = acc + jnp.dot(f0[0:M], w_ref[0, 0], preferred_element_type=jnp.float32)
        acc = acc + jnp.dot(f1[0:M], w_ref[0, 1], preferred_element_type=jnp.float32)
        acc = acc + jnp.dot(f0[Wout:Wout + M], w_ref[0, 2],
                            preferred_element_type=jnp.float32)
        acc = acc + jnp.dot(f1[Wout:Wout + M], w_ref[0, 3],
                            preferred_element_type=jnp.float32)
        acc = jnp.maximum(acc, 0.0)
        if pool:
            y = jnp.maximum(
                jnp.maximum(acc[:, 0:Cout], acc[:, Cout:2 * Cout]),
                jnp.maximum(acc[:, 2 * Cout:3 * Cout], acc[:, 3 * Cout:4 * Cout]))
            o_ref[0] = y.reshape(ts, Wout, Cout).astype(o_ref.dtype)
        else:
            o_ref[0] = acc.reshape(ts, Wout, N4).astype(o_ref.dtype)

    return body


def _conv_layer(xb, w_stack, b_stack, *, batch, Cout, pool, ts, hout=None):
    TB, HB, WB, C4 = xb.shape
    Hout, Wout = (hout if hout is not None else HB - 1), WB - 1
    N4 = 4 * Cout
    Nout = Cout if pool else N4
    assert ts % 8 == 0 and Hout % ts == 0
    n_strips = Hout // ts
    body = _make_conv_body(ts, Wout, C4, Cout, pool)
    return pl.pallas_call(
        body,
        out_shape=jax.ShapeDtypeStruct((TB, Hout, Wout, Nout), jnp.bfloat16),
        grid=(TB, n_strips),
        in_specs=[
            pl.BlockSpec((1, ts, WB, C4), lambda i, r: (i, r, 0, 0)),
            pl.BlockSpec((1, 8, WB, C4), lambda i, r: (i, (r + 1) * ts // 8, 0, 0)),
            pl.BlockSpec((1, 4, C4, N4), lambda i, r: (i // batch, 0, 0, 0)),
            pl.BlockSpec((1, 1, N4), lambda i, r: (i // batch, 0, 0)),
        ],
        out_specs=pl.BlockSpec((1, ts, Wout, Nout), lambda i, r: (i, r, 0, 0)),
        compiler_params=pltpu.CompilerParams(
            dimension_semantics=("parallel", "arbitrary")),
    )(xb, xb, w_stack, b_stack)


def _fc1_body(f_ref, w_ref, b_ref, o_ref, acc_ref):
    k = pl.program_id(1)

    @pl.when(k == 0)
    def _init():
        acc_ref[...] = jnp.broadcast_to(b_ref[0], acc_ref.shape)

    acc_ref[...] += jnp.dot(f_ref[0], w_ref[0].astype(jnp.bfloat16),
                            preferred_element_type=jnp.float32)

    @pl.when(k == pl.num_programs(1) - 1)
    def _finish():
        o_ref[0] = jnp.maximum(acc_ref[...], 0.0).astype(o_ref.dtype)


def _fc1_apply(feats, w, b, *, tk=4096):
    S, B, K = feats.shape
    N = w.shape[-1]
    return pl.pallas_call(
        _fc1_body,
        out_shape=jax.ShapeDtypeStruct((S, B, N), jnp.bfloat16),
        grid=(S, K // tk),
        in_specs=[
            pl.BlockSpec((1, B, tk), lambda s, k: (s, 0, k)),
            pl.BlockSpec((1, tk, N), lambda s, k: (s, k, 0)),
            pl.BlockSpec((1, 1, N), lambda s, k: (s, 0, 0)),
        ],
        out_specs=pl.BlockSpec((1, B, N), lambda s, k: (s, 0, 0)),
        scratch_shapes=[pltpu.VMEM((B, N), jnp.float32)],
        compiler_params=pltpu.CompilerParams(
            dimension_semantics=("parallel", "arbitrary")),
    )(feats, w, b)


def _head_body(f_ref, w1_ref, b1_ref, w2_ref, b2_ref, o_ref):
    w1 = w1_ref[...]
    nh = w1.shape[0] // 2
    hv = (jnp.dot(f_ref[0], w1[0:nh].astype(jnp.bfloat16),
                  preferred_element_type=jnp.float32)
          + jnp.dot(f_ref[1], w1[nh:2 * nh].astype(jnp.bfloat16),
                    preferred_element_type=jnp.float32)
          + b1_ref[...])
    hv = jnp.maximum(hv, 0.0)
    o_ref[...] = jnp.sum(hv * w2_ref[...], axis=1, keepdims=True) + b2_ref[...]


def _head_apply(feats, w1, b1, w2, b2):
    B = feats.shape[1]
    return pl.pallas_call(
        _head_body,
        out_shape=jax.ShapeDtypeStruct((B, 1), jnp.float32),
    )(feats, w1, b1.reshape(1, -1), w2.reshape(1, -1), b2.reshape(1, 1))


def _merge_tower_w(wpair, cin, cout):
    w6 = wpair.reshape(2, 4, 4, cin, 4, cout)
    z = jnp.zeros_like(w6[0])
    top = jnp.concatenate([w6[0], z], axis=4)
    bot = jnp.concatenate([z, w6[1]], axis=4)
    m = jnp.concatenate([top, bot], axis=2)
    return m.reshape(1, 4, 8 * cin, 8 * cout)


def _merge_tower_b(bpair, cout):
    b4 = bpair.reshape(2, 4, cout)
    return jnp.concatenate([b4[0], b4[1]], axis=-1).reshape(1, 1, 8 * cout)


def kernel(x, a, conv21_w, conv21_b,
           lenet_w_0, lenet_w_1, lenet_w_2, lenet_w_3,
           lenet_b_0, lenet_b_1, lenet_b_2, lenet_b_3,
           fc1_w, fc1_b, head_w1, head_b1, head_w2, head_b2):
    B = x.shape[0]
    bf = jnp.bfloat16

    ah = jnp.transpose(a, (0, 2, 3, 1)).astype(bf)
    a21 = _conv_layer(_to_blocks(ah), conv21_w.astype(bf), conv21_b,
                      batch=B, Cout=3, pool=False, ts=32)
    a21 = _blocks_to_image(a21)

    xh = jnp.transpose(x, (0, 2, 3, 1)).astype(bf)
    h = _to_blocks(jnp.concatenate([xh, a21], axis=-1))
    lenet_w = [lenet_w_0, lenet_w_1, lenet_w_2, lenet_w_3]
    lenet_b = [lenet_b_0, lenet_b_1, lenet_b_2, lenet_b_3]
    cfg = [(3, 16, 32), (16, 32, 32), (32, 64, 32), (64, 128, 16)]
    for i, (cin, cout, ts) in enumerate(cfg):
        wm = _merge_tower_w(lenet_w[i].astype(bf), cin, cout)
        bm = _merge_tower_b(lenet_b[i], cout)
        h = _conv_layer(h, wm, bm, batch=B, Cout=2 * cout, pool=True, ts=ts)
        if i + 1 < len(cfg):
            h = _to_blocks(h)

    feats = jnp.stack([h[..., :128].reshape(B, 16 * 16 * 128),
                       h[..., 128:].reshape(B, 16 * 16 * 128)])
    hfc = _fc1_apply(feats, fc1_w, fc1_b)
    return _head_apply(hfc, head_w1, head_b1, head_w2, head_b2)
